# baseline (speedup 1.0000x reference)
import jax
import jax.numpy as jnp
from jax import lax
from jax.experimental import pallas as pl
from jax.experimental.pallas import tpu as pltpu

NUM_CLASSES = 1000
NUM_ROWS = 16384

_BLKC = 8
_GRID = NUM_CLASSES // _BLKC


def _body(x1_ref, out_ref):
    ids = x1_ref[0, 0]
    base = pl.program_id(0) * _BLKC
    cls = lax.broadcasted_iota(jnp.int32, (_BLKC, NUM_ROWS), 0) + base
    out_ref[...] = (cls == ids[None, :]).astype(jnp.float32)


@jax.jit
def kernel(x1):
    x1 = x1.astype(jnp.int32)
    x1r = x1.reshape(1, 1, NUM_ROWS)
    out_t = pl.pallas_call(
        _body,
        grid=(_GRID,),
        in_specs=[pl.BlockSpec((1, 1, NUM_ROWS), lambda i: (0, 0, 0))],
        out_specs=pl.BlockSpec((_BLKC, NUM_ROWS), lambda i: (i, 0)),
        out_shape=jax.ShapeDtypeStruct((NUM_CLASSES, NUM_ROWS), jnp.float32),
    )(x1r)
    return out_t.T


# speedup vs baseline: 2.3684x; 2.3684x over previous
import jax
import jax.numpy as jnp
from jax import lax
from jax.experimental import pallas as pl
from jax.experimental.pallas import tpu as pltpu

NUM_CLASSES = 1000
NUM_ROWS = 16384

_BLKC = 40
_GRID = NUM_CLASSES // _BLKC


def _body(x1_ref, out_ref):
    ids = x1_ref[0, 0]
    base = pl.program_id(0) * _BLKC
    cls = lax.broadcasted_iota(jnp.int32, (_BLKC, NUM_ROWS), 0)
    out_ref[...] = (cls == (ids - base)[None, :]).astype(jnp.float32)


@jax.jit
def kernel(x1):
    x1 = x1.astype(jnp.int32)
    x1r = x1.reshape(1, 1, NUM_ROWS)
    out_t = pl.pallas_call(
        _body,
        grid=(_GRID,),
        in_specs=[pl.BlockSpec((1, 1, NUM_ROWS), lambda i: (0, 0, 0))],
        out_specs=pl.BlockSpec((_BLKC, NUM_ROWS), lambda i: (i, 0)),
        out_shape=jax.ShapeDtypeStruct((NUM_CLASSES, NUM_ROWS), jnp.float32),
    )(x1r)
    return out_t.T


# speedup vs baseline: 2.4096x; 1.0174x over previous
import jax
import jax.numpy as jnp
from jax import lax
from jax.experimental import pallas as pl
from jax.experimental.pallas import tpu as pltpu

NUM_CLASSES = 1000
NUM_ROWS = 16384

_BLKC = 40
_GRID = NUM_CLASSES // _BLKC


def _body(x1_ref, out_ref):
    ids = x1_ref[0, 0]
    base = pl.program_id(0) * _BLKC
    cls = lax.broadcasted_iota(jnp.int32, (_BLKC, NUM_ROWS), 0) + base
    out_ref[...] = (cls == ids[None, :]).astype(jnp.float32)


@jax.jit
def kernel(x1):
    x1 = x1.astype(jnp.int32)
    x1r = x1.reshape(1, 1, NUM_ROWS)
    out_t = pl.pallas_call(
        _body,
        grid=(_GRID,),
        in_specs=[pl.BlockSpec((1, 1, NUM_ROWS), lambda i: (0, 0, 0))],
        out_specs=pl.BlockSpec((_BLKC, NUM_ROWS), lambda i: (i, 0)),
        out_shape=jax.ShapeDtypeStruct((NUM_CLASSES, NUM_ROWS), jnp.float32),
    )(x1r)
    return out_t.T
